# baseline (speedup 1.0000x reference)
import functools

import jax
import jax.numpy as jnp
from jax import lax
from jax.experimental import pallas as pl
from jax.experimental.pallas import tpu as pltpu
from jax.experimental.pallas import tpu_sc as plsc

NUM_CORES = 2
NUM_SUBCORES = 16
LANES = 16
NW = NUM_CORES * NUM_SUBCORES

BATCH = 16384
EMBED_DIM = 16
B_PER_W = BATCH // NW


def _dot_kernel(user_hbm, item_hbm, uemb_hbm, iemb_hbm, out_hbm,
                uidx_v, iidx_v, urows_v, irows_v, out_v, usem, isem):
    wid = lax.axis_index("s") * NUM_CORES + lax.axis_index("c")
    base = wid * B_PER_W

    pltpu.sync_copy(user_hbm.at[pl.ds(base, B_PER_W)], uidx_v)
    pltpu.sync_copy(item_hbm.at[pl.ds(base, B_PER_W)], iidx_v)

    cu = pltpu.async_copy(uemb_hbm.at[uidx_v], urows_v, usem)
    ci = pltpu.async_copy(iemb_hbm.at[iidx_v], irows_v, isem)
    cu.wait()
    ci.wait()

    zero = jnp.zeros((LANES,), jnp.float32)

    def zbody(g, carry):
        out_v[pl.ds(g * LANES, LANES)] = zero
        return carry

    lax.fori_loop(0, B_PER_W // LANES, zbody, 0)

    def body(r, carry):
        p = urows_v[r, :] * irows_v[r, :]
        ridx = jnp.full((LANES,), 0, jnp.int32) + r
        plsc.addupdate_scatter(out_v, [ridx], p)
        return carry

    lax.fori_loop(0, B_PER_W, body, 0, unroll=4)

    pltpu.sync_copy(out_v, out_hbm.at[pl.ds(base, B_PER_W)])


@jax.jit
def kernel(user, item, user_emb, item_emb):
    mesh = plsc.VectorSubcoreMesh(
        core_axis_name="c", subcore_axis_name="s",
        num_cores=NUM_CORES, num_subcores=NUM_SUBCORES)
    run = pl.kernel(
        _dot_kernel,
        out_type=jax.ShapeDtypeStruct((BATCH,), jnp.float32),
        mesh=mesh,
        compiler_params=pltpu.CompilerParams(
            needs_layout_passes=False, use_tc_tiling_on_sc=False),
        scratch_types=[
            pltpu.VMEM((B_PER_W,), jnp.int32),
            pltpu.VMEM((B_PER_W,), jnp.int32),
            pltpu.VMEM((B_PER_W, EMBED_DIM), jnp.float32),
            pltpu.VMEM((B_PER_W, EMBED_DIM), jnp.float32),
            pltpu.VMEM((B_PER_W,), jnp.float32),
            pltpu.SemaphoreType.DMA,
            pltpu.SemaphoreType.DMA,
        ],
    )
    return run(user, item, user_emb, item_emb)
